# baseline (speedup 1.0000x reference)
import jax
import jax.numpy as jnp
from jax import lax
from jax.experimental import pallas as pl
from jax.experimental.pallas import tpu as pltpu

NUM_TOKENS = 32768
HIDDEN = 4096
ROUTER_HIDDEN = 512
NUM_EXPERTS = 64
TOP_K = 8
TM = 1024


NSPLIT = 1
HALF = TM // NSPLIT


def _topk_from_logits(logits, rows):
    bits = lax.bitcast_convert_type(logits, jnp.int32)
    iota = lax.broadcasted_iota(jnp.int32, (rows, NUM_EXPERTS), 1)
    mlow = jnp.where(bits >= 0, jnp.int32(63) - iota, iota)
    keys = lax.bitcast_convert_type((bits & jnp.int32(~0x3F)) | mlow,
                                    jnp.float32)
    key_cols = []
    for k in range(TOP_K):
        m = jnp.max(keys, axis=-1, keepdims=True)
        key_cols.append(m)
        if k < TOP_K - 1:
            keys = jnp.where(keys == m, -jnp.inf, keys)
    top_keys = lax.bitcast_convert_type(
        jnp.concatenate(key_cols, axis=1), jnp.int32)
    top_idx = jnp.where(top_keys >= 0,
                        jnp.int32(63) - (top_keys & jnp.int32(0x3F)),
                        top_keys & jnp.int32(0x3F))
    top_vals = lax.bitcast_convert_type(top_keys & jnp.int32(~0x3F),
                                        jnp.float32)
    e = jnp.exp(top_vals - top_vals[:, 0:1])
    return e / jnp.sum(e, axis=-1, keepdims=True), top_idx


def _router_kernel(x_ref, w1_ref, b1_ref, w2_ref, b2_ref, out_w_ref, out_i_ref):
    for s in range(NSPLIT):
        rows = pl.ds(s * HALF, HALF)
        x = x_ref[rows, :]
        h = jnp.dot(x, w1_ref[...], preferred_element_type=jnp.float32)
        h = jnp.maximum(h + b1_ref[...], 0.0)
        logits = jnp.dot(h, w2_ref[...], preferred_element_type=jnp.float32)
        logits = logits + b2_ref[...]
        w, idx = _topk_from_logits(logits, HALF)
        out_w_ref[rows, :] = w
        out_i_ref[rows, :] = idx


def kernel(hidden_states, W1, b1, W2, b2):
    grid = (NUM_TOKENS // TM,)
    out = pl.pallas_call(
        _router_kernel,
        grid=grid,
        in_specs=[
            pl.BlockSpec((TM, HIDDEN), lambda i: (i, 0)),
            pl.BlockSpec((HIDDEN, ROUTER_HIDDEN), lambda i: (0, 0)),
            pl.BlockSpec((1, ROUTER_HIDDEN), lambda i: (0, 0)),
            pl.BlockSpec((ROUTER_HIDDEN, NUM_EXPERTS), lambda i: (0, 0)),
            pl.BlockSpec((1, NUM_EXPERTS), lambda i: (0, 0)),
        ],
        out_specs=[
            pl.BlockSpec((TM, TOP_K), lambda i: (i, 0)),
            pl.BlockSpec((TM, TOP_K), lambda i: (i, 0)),
        ],
        out_shape=[
            jax.ShapeDtypeStruct((NUM_TOKENS, TOP_K), jnp.float32),
            jax.ShapeDtypeStruct((NUM_TOKENS, TOP_K), jnp.int32),
        ],
        compiler_params=pltpu.CompilerParams(
            dimension_semantics=("parallel",),
        ),
    )(
        hidden_states,
        W1.astype(jnp.bfloat16),
        b1.reshape(1, ROUTER_HIDDEN),
        W2.astype(jnp.bfloat16),
        b2.reshape(1, NUM_EXPERTS),
    )
    return (out[0], out[1])


# speedup vs baseline: 1.1748x; 1.1748x over previous
import jax
import jax.numpy as jnp
from jax import lax
from jax.experimental import pallas as pl
from jax.experimental.pallas import tpu as pltpu

NUM_TOKENS = 32768
HIDDEN = 4096
ROUTER_HIDDEN = 512
NUM_EXPERTS = 64
TOP_K = 8
TM = 1024


NSPLIT = 2
HALF = TM // NSPLIT


def _topk_from_logits(logits, rows):
    bits = lax.bitcast_convert_type(logits, jnp.int32)
    iota = lax.broadcasted_iota(jnp.int32, (rows, NUM_EXPERTS), 1)
    mlow = jnp.where(bits >= 0, jnp.int32(63) - iota, iota)
    keys = lax.bitcast_convert_type((bits & jnp.int32(~0x3F)) | mlow,
                                    jnp.float32)
    key_cols = []
    for k in range(TOP_K):
        m = jnp.max(keys, axis=-1, keepdims=True)
        key_cols.append(m)
        if k < TOP_K - 1:
            keys = jnp.where(keys == m, -jnp.inf, keys)
    top_keys = lax.bitcast_convert_type(
        jnp.concatenate(key_cols, axis=1), jnp.int32)
    top_idx = jnp.where(top_keys >= 0,
                        jnp.int32(63) - (top_keys & jnp.int32(0x3F)),
                        top_keys & jnp.int32(0x3F))
    top_vals = lax.bitcast_convert_type(top_keys & jnp.int32(~0x3F),
                                        jnp.float32)
    e = jnp.exp(top_vals - top_vals[:, 0:1])
    return e / jnp.sum(e, axis=-1, keepdims=True), top_idx


def _router_kernel(x_ref, w1_ref, b1_ref, w2_ref, b2_ref, out_w_ref, out_i_ref):
    xb = x_ref[...].astype(jnp.bfloat16)
    for s in range(NSPLIT):
        rows = pl.ds(s * HALF, HALF)
        x = xb[s * HALF:(s + 1) * HALF, :]
        h = jnp.dot(x, w1_ref[...], preferred_element_type=jnp.float32)
        h = jnp.maximum(h + b1_ref[...], 0.0).astype(jnp.bfloat16)
        logits = jnp.dot(h, w2_ref[...], preferred_element_type=jnp.float32)
        logits = logits + b2_ref[...]
        w, idx = _topk_from_logits(logits, HALF)
        out_w_ref[rows, :] = w
        out_i_ref[rows, :] = idx


def kernel(hidden_states, W1, b1, W2, b2):
    grid = (NUM_TOKENS // TM,)
    out = pl.pallas_call(
        _router_kernel,
        grid=grid,
        in_specs=[
            pl.BlockSpec((TM, HIDDEN), lambda i: (i, 0)),
            pl.BlockSpec((HIDDEN, ROUTER_HIDDEN), lambda i: (0, 0)),
            pl.BlockSpec((1, ROUTER_HIDDEN), lambda i: (0, 0)),
            pl.BlockSpec((ROUTER_HIDDEN, NUM_EXPERTS), lambda i: (0, 0)),
            pl.BlockSpec((1, NUM_EXPERTS), lambda i: (0, 0)),
        ],
        out_specs=[
            pl.BlockSpec((TM, TOP_K), lambda i: (i, 0)),
            pl.BlockSpec((TM, TOP_K), lambda i: (i, 0)),
        ],
        out_shape=[
            jax.ShapeDtypeStruct((NUM_TOKENS, TOP_K), jnp.float32),
            jax.ShapeDtypeStruct((NUM_TOKENS, TOP_K), jnp.int32),
        ],
        compiler_params=pltpu.CompilerParams(
            dimension_semantics=("parallel",),
        ),
    )(
        hidden_states,
        W1.astype(jnp.bfloat16),
        b1.reshape(1, ROUTER_HIDDEN),
        W2.astype(jnp.bfloat16),
        b2.reshape(1, NUM_EXPERTS),
    )
    return (out[0], out[1])
